# Initial kernel scaffold; baseline (speedup 1.0000x reference)
import jax
import jax.numpy as jnp
from jax.experimental import pallas as pl


def kernel(x, Wq, bq, Wk, bk, Wv, bv, Wo, bo, probes):
    raise NotImplementedError("write your pallas kernel here")


# speedup vs baseline: 6.5667x; 6.5667x over previous
import functools
import math

import numpy as np
import jax
import jax.numpy as jnp
from jax import lax
from jax.experimental import pallas as pl
from jax.experimental.pallas import tpu as pltpu
from jax.experimental.pallas import tpu_sc as plsc

B, N, E = 2, 8192, 768
H = 12
Dh = E // H
P = 64
BUCKET_SIZES = (256, 64, 16)
NL = len(BUCKET_SIZES)
BH = B * H
PK = 4 * Dh
OW = 2 * Dh

NC, NS = 2, 16
NW = NC * NS
CHUNK = 128
NCK = N // CHUNK

CSB = 512


def _class_matrix(n_buckets: int) -> np.ndarray:
    bv = [(r * 9973) % n_buckets for r in range(P)]
    distinct = sorted(set(bv))
    cls = {v: i for i, v in enumerate(distinct)}
    m = np.zeros((P, P), np.float32)
    for r in range(P):
        m[r, cls[bv[r]]] = 1.0
    return m


QKV_RB = 512


def _qkv_body(x_ref, wq_ref, wk_ref, wv_ref, bqkv_ref, pk_ref, q_ref):
    xb = x_ref[0]
    ts = []
    for w_ref, bi in ((wq_ref, 0), (wk_ref, 1), (wv_ref, 2)):
        ts.append(jnp.dot(xb, w_ref[...],
                          preferred_element_type=jnp.float32)
                  + bqkv_ref[bi:bi + 1, :])
    tq, tk, tv = ts
    zpad = jnp.zeros((QKV_RB, Dh), jnp.float32)
    for h in range(H):
        sl = slice(h * Dh, (h + 1) * Dh)
        pk_ref[0, h] = jnp.concatenate(
            [tq[:, sl], tk[:, sl], tv[:, sl], zpad], axis=1)
        q_ref[0, h] = tq[:, sl]


def _qkv(x3, wqT, wkT, wvT, bqkv):
    grid = (B, N // QKV_RB)
    wspec = pl.BlockSpec((E, E), lambda b, i: (0, 0))
    xs = pl.BlockSpec((1, QKV_RB, E), lambda b, i: (b, i, 0))
    return pl.pallas_call(
        _qkv_body,
        grid=grid,
        in_specs=[xs, wspec, wspec, wspec,
                  pl.BlockSpec((NL, E), lambda b, i: (0, 0))],
        out_specs=[
            pl.BlockSpec((1, H, QKV_RB, PK), lambda b, i: (b, 0, i, 0)),
            pl.BlockSpec((1, H, QKV_RB, Dh), lambda b, i: (b, 0, i, 0)),
        ],
        out_shape=[
            jax.ShapeDtypeStruct((B, H, N, PK), jnp.float32),
            jax.ShapeDtypeStruct((B, H, N, Dh), jnp.float32),
        ],
    )(x3, wqT, wkT, wvT, bqkv)


def _rank_body(q_ref, probes_ref, m_ref, tril_ref, su_ref, ranks_ref,
               ohc_ref, run_ref):
    b = pl.program_id(0)
    h = pl.program_id(1)
    q2 = q_ref[0, 0]
    lane = lax.broadcasted_iota(jnp.int32, (N, P), 1)
    rows = []
    for lvl in range(NL):
        proj = jnp.dot(q2, probes_ref[lvl, 0],
                       preferred_element_type=jnp.float32)
        mx = jnp.max(proj, axis=1, keepdims=True)
        region = jnp.min(jnp.where(proj == mx, lane, P), axis=1,
                         keepdims=True)
        oh_r = (region == lane).astype(jnp.bfloat16)
        ohc_ref[...] = jnp.dot(oh_r, m_ref[lvl].astype(jnp.bfloat16),
                               preferred_element_type=jnp.float32)

        def blk(i, carry):
            seg = ohc_ref[pl.ds(i * CSB, CSB), :]
            run = jnp.dot(tril_ref[...], seg.astype(jnp.bfloat16),
                          preferred_element_type=jnp.float32) + carry
            run_ref[pl.ds(i * CSB, CSB), :] = run
            return run[CSB - 1:CSB, :]

        totals = lax.fori_loop(0, N // CSB, blk,
                               jnp.zeros((1, P), jnp.float32))
        offs = jnp.dot(totals, su_ref[...], precision="highest",
                       preferred_element_type=jnp.float32)
        vals = run_ref[...] + (offs - 1.0)
        rankf = jnp.sum(ohc_ref[...] * vals, axis=1)
        rank = rankf.astype(jnp.int32) + (b * H + h) * N
        rows.append(rank.reshape(1, N))
    ranks_ref[0, 0] = jnp.concatenate(rows, axis=0)


def _ranks(q4, probes, mcls, tril, su):
    return pl.pallas_call(
        _rank_body,
        grid=(B, H),
        in_specs=[
            pl.BlockSpec((1, 1, N, Dh), lambda b, h: (b, h, 0, 0)),
            pl.BlockSpec((NL, 1, Dh, P), lambda b, h: (0, h, 0, 0)),
            pl.BlockSpec((NL, P, P), lambda b, h: (0, 0, 0)),
            pl.BlockSpec((CSB, CSB), lambda b, h: (0, 0)),
            pl.BlockSpec((P, P), lambda b, h: (0, 0)),
        ],
        out_specs=pl.BlockSpec((1, 1, NL, N), lambda b, h: (b, h, 0, 0)),
        out_shape=jax.ShapeDtypeStruct((B, H, NL, N), jnp.int32),
        scratch_shapes=[
            pltpu.VMEM((N, P), jnp.float32),
            pltpu.VMEM((N, P), jnp.float32),
        ],
    )(q4, probes, mcls, tril, su)


def _sc_sort(ranks_flat, pk2):
    mesh = plsc.VectorSubcoreMesh(core_axis_name="c", subcore_axis_name="s")
    n_per = BH * NCK // NW

    @functools.partial(
        pl.kernel, mesh=mesh,
        out_type=[jax.ShapeDtypeStruct((BH * N, PK), jnp.float32)] * NL,
        scratch_types=[
            pltpu.VMEM((CHUNK,), jnp.int32),
            pltpu.VMEM((CHUNK, PK), jnp.float32),
            pltpu.SemaphoreType.DMA,
        ],
    )
    def body(ranks_h, pk_h, s0, s1, s2, idx_v, rows, sem):
        wid = lax.axis_index("s") * NC + lax.axis_index("c")
        outs = (s0, s1, s2)

        for lvl in range(NL):
            dst = outs[lvl]

            def task(i, _, lvl=lvl, dst=dst):
                t = wid * n_per + i
                bh = t // NCK
                ck = t - bh * NCK
                n0 = ck * CHUNK
                row0 = bh * N + n0
                roff = (bh * NL + lvl) * N + n0
                pltpu.sync_copy(ranks_h.at[pl.ds(roff, CHUNK)], idx_v)
                pltpu.sync_copy(pk_h.at[pl.ds(row0, CHUNK)], rows)
                pltpu.async_copy(rows, dst.at[idx_v], sem).wait()
                return 0

            lax.fori_loop(0, n_per, task, 0)

    return body(ranks_flat, pk2)


def _att_body(cs, r, m_ref, halo_ref, o_ref):
    g = pl.program_id(1)
    qb = m_ref[0, :, 0, :]
    kwin = jnp.concatenate([halo_ref[0, :, 1, :], m_ref[0, :, 1, :]],
                           axis=0)
    vwin = jnp.concatenate([halo_ref[0, :, 2, :], m_ref[0, :, 2, :]],
                           axis=0)
    scores = lax.dot_general(
        qb, kwin, (((1,), (1,)), ((), ())),
        preferred_element_type=jnp.float32) * (1.0 / math.sqrt(Dh))
    ci = lax.broadcasted_iota(jnp.int32, (r, r + cs), 0) // cs
    kc = lax.broadcasted_iota(jnp.int32, (r, r + cs), 1) // cs - 1
    valid = (kc == ci) | (kc == ci - 1)
    valid &= ~((g == 0) & (kc == -1))
    scores = jnp.where(valid, scores, scores - 1e9)
    m = jnp.max(scores, axis=1, keepdims=True)
    e = jnp.exp(scores - m)
    s = jnp.sum(e, axis=1, keepdims=True)
    attn = e / s
    out = lax.dot_general(attn, vwin, (((1,), (0,)), ((), ())),
                          preferred_element_type=jnp.float32)
    o_ref[0] = jnp.concatenate(
        [out, jnp.zeros((r, OW - Dh), jnp.float32)], axis=1)


def _attention(cs, r, sorted_pk):
    grid = (BH, N // r)
    gg = r // cs
    mspec = pl.BlockSpec((1, r, 4, Dh), lambda bh, g: (bh, g, 0, 0))
    halo = pl.BlockSpec((1, cs, 4, Dh),
                        lambda bh, g: (bh, jnp.maximum(g * gg - 1, 0), 0, 0))
    out = pl.pallas_call(
        functools.partial(_att_body, cs, r),
        grid=grid,
        in_specs=[mspec, halo],
        out_specs=pl.BlockSpec((1, r, OW), lambda bh, g: (bh, g, 0)),
        out_shape=jax.ShapeDtypeStruct((BH, N, OW), jnp.float32),
    )(sorted_pk.reshape(BH, N, 4, Dh), sorted_pk.reshape(BH, N, 4, Dh))
    return out.reshape(BH * N, OW)


def _sc_unsort(ranks_flat, o0, o1, o2):
    mesh = plsc.VectorSubcoreMesh(core_axis_name="c", subcore_axis_name="s")
    n_per = BH * NCK // NW

    @functools.partial(
        pl.kernel, mesh=mesh,
        out_type=jax.ShapeDtypeStruct((BH * N, OW), jnp.float32),
        scratch_types=[
            pltpu.VMEM((CHUNK,), jnp.int32),
            pltpu.VMEM((CHUNK,), jnp.int32),
            pltpu.VMEM((CHUNK,), jnp.int32),
            pltpu.VMEM((CHUNK, OW), jnp.float32),
            pltpu.VMEM((CHUNK, OW), jnp.float32),
            pltpu.VMEM((CHUNK, OW), jnp.float32),
            pltpu.VMEM((CHUNK,), jnp.int32),
            pltpu.VMEM_SHARED((NS * CHUNK, OW), jnp.float32),
            pltpu.SemaphoreType.DMA,
        ],
    )
    def body(ranks_h, o0_h, o1_h, o2_h, merged_h,
             i0, i1, i2, r0, r1, r2, lin, acc_sh, sem):
        wid = lax.axis_index("s") * NC + lax.axis_index("c")
        sid = lax.axis_index("s")
        for j in range(CHUNK // 16):
            lin[pl.ds(j * 16, 16)] = (
                lax.broadcasted_iota(jnp.int32, (16,), 0)
                + (sid * CHUNK + j * 16))

        def task(i, _):
            t = wid * n_per + i
            bh = t // NCK
            ck = t - bh * NCK
            n0 = ck * CHUNK
            roff = bh * NL * N + n0
            pltpu.sync_copy(ranks_h.at[pl.ds(roff, CHUNK)], i0)
            pltpu.sync_copy(ranks_h.at[pl.ds(roff + N, CHUNK)], i1)
            pltpu.sync_copy(ranks_h.at[pl.ds(roff + 2 * N, CHUNK)], i2)
            c0 = pltpu.async_copy(o0_h.at[i0], r0, sem)
            c1 = pltpu.async_copy(o1_h.at[i1], r1, sem)
            c2 = pltpu.async_copy(o2_h.at[i2], r2, sem)
            c0.wait(); c1.wait(); c2.wait()
            pltpu.sync_copy(r0, acc_sh.at[pl.ds(sid * CHUNK, CHUNK)])
            pltpu.sync_copy(r1, acc_sh.at[lin], add=True)
            pltpu.sync_copy(r2, acc_sh.at[lin], add=True)
            pltpu.sync_copy(acc_sh.at[pl.ds(sid * CHUNK, CHUNK)],
                            merged_h.at[pl.ds(bh * N + n0, CHUNK)])
            return 0

        lax.fori_loop(0, n_per, task, 0)

    return body(ranks_flat, o0, o1, o2)


def _out_body(m_ref, woT_ref, bo_ref, o_ref):
    m2 = jnp.concatenate([m_ref[0, h, :, :Dh] for h in range(H)],
                         axis=1)
    o_ref[0] = (
        jnp.dot(m2, woT_ref[...], precision="highest",
                preferred_element_type=jnp.float32) + bo_ref[...])


def _oproj(merged, woT3, bo):
    rb = 512
    out = pl.pallas_call(
        _out_body,
        grid=(B, N // rb),
        in_specs=[pl.BlockSpec((1, H, rb, OW), lambda b, i: (b, 0, i, 0)),
                  pl.BlockSpec((E, E), lambda b, i: (0, 0)),
                  pl.BlockSpec((1, E), lambda b, i: (0, 0))],
        out_specs=pl.BlockSpec((1, rb, E), lambda b, i: (b, i, 0)),
        out_shape=jax.ShapeDtypeStruct((B, N, E), jnp.float32),
    )(merged.reshape(B, H, N, OW), woT3, bo.reshape(1, E))
    return out


def kernel(x, Wq, bq, Wk, bk, Wv, bv, Wo, bo, probes):
    bqkv = jnp.stack([bq, bk, bv], axis=0)
    pk4, q4 = _qkv(x, Wq.T, Wk.T, Wv.T, bqkv)

    mcls = jnp.asarray(np.stack(
        [_class_matrix(max(1, N // cs)) for cs in BUCKET_SIZES]))
    tril = jnp.asarray(np.tril(np.ones((CSB, CSB), np.float32))
                       .astype(np.float32)).astype(jnp.bfloat16)
    su = jnp.asarray(np.triu(np.ones((P, P), np.float32), 1))
    ranks = _ranks(q4, probes, mcls, tril, su)
    ranks_flat = ranks.reshape(-1)

    sorted_all = _sc_sort(ranks_flat, pk4.reshape(BH * N, PK))
    outs = []
    rblocks = {256: 1024, 64: 512, 16: 256}
    for lvl, cs in enumerate(BUCKET_SIZES):
        outs.append(_attention(cs, rblocks[cs], sorted_all[lvl]))

    merged = _sc_unsort(ranks_flat, *outs)
    return _oproj(merged, Wo.T * (1.0 / NL), bo)


# speedup vs baseline: 8.0565x; 1.2269x over previous
import functools
import math

import numpy as np
import jax
import jax.numpy as jnp
from jax import lax
from jax.experimental import pallas as pl
from jax.experimental.pallas import tpu as pltpu
from jax.experimental.pallas import tpu_sc as plsc

B, N, E = 2, 8192, 768
H = 12
Dh = E // H
P = 64
BUCKET_SIZES = (256, 64, 16)
NL = len(BUCKET_SIZES)
BH = B * H
PK = 4 * Dh
OW = 2 * Dh

NC, NS = 2, 16
NW = NC * NS
CHUNK = 128
NCK = N // CHUNK

CSB = 512


def _class_matrix(n_buckets: int) -> np.ndarray:
    bv = [(r * 9973) % n_buckets for r in range(P)]
    distinct = sorted(set(bv))
    cls = {v: i for i, v in enumerate(distinct)}
    m = np.zeros((P, P), np.float32)
    for r in range(P):
        m[r, cls[bv[r]]] = 1.0
    return m


QKV_RB = 512


def _qkv_body(x_ref, wq_ref, wk_ref, wv_ref, bqkv_ref, pk_ref, q_ref):
    xb = x_ref[0]
    ts = []
    for w_ref, bi in ((wq_ref, 0), (wk_ref, 1), (wv_ref, 2)):
        ts.append(jnp.dot(xb, w_ref[...],
                          preferred_element_type=jnp.float32)
                  + bqkv_ref[bi:bi + 1, :])
    tq, tk, tv = ts
    zpad = jnp.zeros((QKV_RB, Dh), jnp.float32)
    for h in range(H):
        sl = slice(h * Dh, (h + 1) * Dh)
        pk_ref[0, h] = jnp.concatenate(
            [tq[:, sl], tk[:, sl], tv[:, sl], zpad], axis=1)
        q_ref[0, h] = tq[:, sl]


def _qkv(x3, wqT, wkT, wvT, bqkv):
    grid = (B, N // QKV_RB)
    wspec = pl.BlockSpec((E, E), lambda b, i: (0, 0))
    xs = pl.BlockSpec((1, QKV_RB, E), lambda b, i: (b, i, 0))
    return pl.pallas_call(
        _qkv_body,
        grid=grid,
        in_specs=[xs, wspec, wspec, wspec,
                  pl.BlockSpec((NL, E), lambda b, i: (0, 0))],
        out_specs=[
            pl.BlockSpec((1, H, QKV_RB, PK), lambda b, i: (b, 0, i, 0)),
            pl.BlockSpec((1, H, QKV_RB, Dh), lambda b, i: (b, 0, i, 0)),
        ],
        out_shape=[
            jax.ShapeDtypeStruct((B, H, N, PK), jnp.float32),
            jax.ShapeDtypeStruct((B, H, N, Dh), jnp.float32),
        ],
    )(x3, wqT, wkT, wvT, bqkv)


LW = NL * P


def _rank_body(q_ref, probes_ref, m3_ref, tril_ref, su3_ref, ranks_ref,
               ohc_ref, run_ref):
    b = pl.program_id(0)
    h = pl.program_id(1)
    q2 = q_ref[0, 0]
    probes3 = jnp.concatenate([probes_ref[lvl, 0] for lvl in range(NL)],
                              axis=1)
    proj = jnp.dot(q2, probes3,
                   preferred_element_type=jnp.float32)
    lane = lax.broadcasted_iota(jnp.int32, (N, P), 1)
    ohs = []
    for lvl in range(NL):
        pr = proj[:, lvl * P:(lvl + 1) * P]
        mx = jnp.max(pr, axis=1, keepdims=True)
        region = jnp.min(jnp.where(pr == mx, lane, P), axis=1,
                         keepdims=True)
        ohs.append((region == lane).astype(jnp.bfloat16))
    oh3 = jnp.concatenate(ohs, axis=1)
    ohc_ref[...] = jnp.dot(oh3, m3_ref[...],
                           preferred_element_type=jnp.float32)

    def blk(i, carry):
        seg = ohc_ref[pl.ds(i * CSB, CSB), :]
        run = jnp.dot(tril_ref[...], seg.astype(jnp.bfloat16),
                      preferred_element_type=jnp.float32) + carry
        run_ref[pl.ds(i * CSB, CSB), :] = run
        return run[CSB - 1:CSB, :]

    totals = lax.fori_loop(0, N // CSB, blk,
                           jnp.zeros((1, LW), jnp.float32))
    offs = jnp.dot(totals, su3_ref[...], precision="highest",
                   preferred_element_type=jnp.float32)
    vals = ohc_ref[...] * (run_ref[...] + (offs - 1.0))
    base = (b * H + h) * N
    rows = []
    for lvl in range(NL):
        rankf = jnp.sum(vals[:, lvl * P:(lvl + 1) * P], axis=1)
        rows.append((rankf.astype(jnp.int32) + base).reshape(1, N))
    ranks_ref[0, 0] = jnp.concatenate(rows, axis=0)


def _ranks(q4, probes, m3, tril, su3):
    return pl.pallas_call(
        _rank_body,
        grid=(B, H),
        in_specs=[
            pl.BlockSpec((1, 1, N, Dh), lambda b, h: (b, h, 0, 0)),
            pl.BlockSpec((NL, 1, Dh, P), lambda b, h: (0, h, 0, 0)),
            pl.BlockSpec((LW, LW), lambda b, h: (0, 0)),
            pl.BlockSpec((CSB, CSB), lambda b, h: (0, 0)),
            pl.BlockSpec((LW, LW), lambda b, h: (0, 0)),
        ],
        out_specs=pl.BlockSpec((1, 1, NL, N), lambda b, h: (b, h, 0, 0)),
        out_shape=jax.ShapeDtypeStruct((B, H, NL, N), jnp.int32),
        scratch_shapes=[
            pltpu.VMEM((N, LW), jnp.float32),
            pltpu.VMEM((N, LW), jnp.float32),
        ],
    )(q4, probes, m3, tril, su3)


def _sc_sort(ranks_flat, pk2):
    mesh = plsc.VectorSubcoreMesh(core_axis_name="c", subcore_axis_name="s")
    n_per = BH * NCK // NW

    @functools.partial(
        pl.kernel, mesh=mesh,
        out_type=[jax.ShapeDtypeStruct((BH * N, PK), jnp.float32)] * NL,
        scratch_types=[
            pltpu.VMEM((CHUNK,), jnp.int32),
            pltpu.VMEM((CHUNK, PK), jnp.float32),
            pltpu.SemaphoreType.DMA,
        ],
    )
    def body(ranks_h, pk_h, s0, s1, s2, idx_v, rows, sem):
        wid = lax.axis_index("s") * NC + lax.axis_index("c")
        outs = (s0, s1, s2)

        for lvl in range(NL):
            dst = outs[lvl]

            def task(i, _, lvl=lvl, dst=dst):
                t = wid * n_per + i
                bh = t // NCK
                ck = t - bh * NCK
                n0 = ck * CHUNK
                row0 = bh * N + n0
                roff = (bh * NL + lvl) * N + n0
                pltpu.sync_copy(ranks_h.at[pl.ds(roff, CHUNK)], idx_v)
                pltpu.sync_copy(pk_h.at[pl.ds(row0, CHUNK)], rows)
                pltpu.async_copy(rows, dst.at[idx_v], sem).wait()
                return 0

            lax.fori_loop(0, n_per, task, 0)

    return body(ranks_flat, pk2)


def _att_body(cs, r, sb, m_ref, halo_ref, o_ref):
    g = pl.program_id(1)
    nb = r // sb
    kb = sb + cs
    qb = m_ref[0, :, 0, :].astype(jnp.bfloat16)
    kwin = jnp.concatenate(
        [halo_ref[0, :, 1, :], m_ref[0, :, 1, :]],
        axis=0).astype(jnp.bfloat16)
    vwin = jnp.concatenate(
        [halo_ref[0, :, 2, :], m_ref[0, :, 2, :]],
        axis=0).astype(jnp.bfloat16)
    q3 = qb.reshape(nb, sb, Dh)
    k3 = jnp.concatenate(
        [kwin[s * sb:s * sb + kb].reshape(1, kb, Dh) for s in range(nb)],
        axis=0)
    v3 = jnp.concatenate(
        [vwin[s * sb:s * sb + kb].reshape(1, kb, Dh) for s in range(nb)],
        axis=0)
    scores = lax.dot_general(
        q3, k3, (((2,), (2,)), ((0,), (0,))),
        preferred_element_type=jnp.float32) * (1.0 / math.sqrt(Dh))
    ci = lax.broadcasted_iota(jnp.int32, (nb, sb, kb), 1) // cs
    kc = lax.broadcasted_iota(jnp.int32, (nb, sb, kb), 2) // cs - 1
    valid = (kc == ci) | (kc == ci - 1)
    si = lax.broadcasted_iota(jnp.int32, (nb, sb, kb), 0)
    valid &= ~((g == 0) & (si == 0) & (kc == -1))
    scores = jnp.where(valid, scores, scores - 1e9)
    m = jnp.max(scores, axis=2, keepdims=True)
    e = jnp.exp(scores - m)
    s = jnp.sum(e, axis=2, keepdims=True)
    attn = (e / s).astype(jnp.bfloat16)
    out = lax.dot_general(attn, v3, (((2,), (1,)), ((0,), (0,))),
                          preferred_element_type=jnp.float32)
    o_ref[0] = jnp.concatenate(
        [out.reshape(r, Dh), jnp.zeros((r, OW - Dh), jnp.float32)], axis=1)


def _attention(cs, r, sb, sorted_pk):
    grid = (BH, N // r)
    gg = r // cs
    mspec = pl.BlockSpec((1, r, 4, Dh), lambda bh, g: (bh, g, 0, 0))
    halo = pl.BlockSpec((1, cs, 4, Dh),
                        lambda bh, g: (bh, jnp.maximum(g * gg - 1, 0), 0, 0))
    out = pl.pallas_call(
        functools.partial(_att_body, cs, r, sb),
        grid=grid,
        in_specs=[mspec, halo],
        out_specs=pl.BlockSpec((1, r, OW), lambda bh, g: (bh, g, 0)),
        out_shape=jax.ShapeDtypeStruct((BH, N, OW), jnp.float32),
    )(sorted_pk.reshape(BH, N, 4, Dh), sorted_pk.reshape(BH, N, 4, Dh))
    return out.reshape(BH * N, OW)


def _sc_unsort(ranks_flat, o0, o1, o2):
    mesh = plsc.VectorSubcoreMesh(core_axis_name="c", subcore_axis_name="s")
    n_per = BH * NCK // NW

    @functools.partial(
        pl.kernel, mesh=mesh,
        out_type=jax.ShapeDtypeStruct((BH * N, OW), jnp.float32),
        scratch_types=[
            pltpu.VMEM((CHUNK,), jnp.int32),
            pltpu.VMEM((CHUNK,), jnp.int32),
            pltpu.VMEM((CHUNK,), jnp.int32),
            pltpu.VMEM((CHUNK, OW), jnp.float32),
            pltpu.VMEM((CHUNK, OW), jnp.float32),
            pltpu.VMEM((CHUNK, OW), jnp.float32),
            pltpu.VMEM((CHUNK,), jnp.int32),
            pltpu.VMEM_SHARED((NS * CHUNK, OW), jnp.float32),
            pltpu.SemaphoreType.DMA,
        ],
    )
    def body(ranks_h, o0_h, o1_h, o2_h, merged_h,
             i0, i1, i2, r0, r1, r2, lin, acc_sh, sem):
        wid = lax.axis_index("s") * NC + lax.axis_index("c")
        sid = lax.axis_index("s")
        for j in range(CHUNK // 16):
            lin[pl.ds(j * 16, 16)] = (
                lax.broadcasted_iota(jnp.int32, (16,), 0)
                + (sid * CHUNK + j * 16))

        def task(i, _):
            t = wid * n_per + i
            bh = t // NCK
            ck = t - bh * NCK
            n0 = ck * CHUNK
            roff = bh * NL * N + n0
            pltpu.sync_copy(ranks_h.at[pl.ds(roff, CHUNK)], i0)
            pltpu.sync_copy(ranks_h.at[pl.ds(roff + N, CHUNK)], i1)
            pltpu.sync_copy(ranks_h.at[pl.ds(roff + 2 * N, CHUNK)], i2)
            c0 = pltpu.async_copy(o0_h.at[i0], r0, sem)
            c1 = pltpu.async_copy(o1_h.at[i1], r1, sem)
            c2 = pltpu.async_copy(o2_h.at[i2], r2, sem)
            c0.wait(); c1.wait(); c2.wait()
            pltpu.sync_copy(r0, acc_sh.at[pl.ds(sid * CHUNK, CHUNK)])
            pltpu.sync_copy(r1, acc_sh.at[lin], add=True)
            pltpu.sync_copy(r2, acc_sh.at[lin], add=True)
            pltpu.sync_copy(acc_sh.at[pl.ds(sid * CHUNK, CHUNK)],
                            merged_h.at[pl.ds(bh * N + n0, CHUNK)])
            return 0

        lax.fori_loop(0, n_per, task, 0)

    return body(ranks_flat, o0, o1, o2)


def _out_body(m_ref, woT_ref, bo_ref, o_ref):
    m2 = jnp.concatenate([m_ref[0, h, :, :Dh] for h in range(H)],
                         axis=1)
    o_ref[0] = (
        jnp.dot(m2, woT_ref[...], precision="highest",
                preferred_element_type=jnp.float32) + bo_ref[...])


def _oproj(merged, woT3, bo):
    rb = 512
    out = pl.pallas_call(
        _out_body,
        grid=(B, N // rb),
        in_specs=[pl.BlockSpec((1, H, rb, OW), lambda b, i: (b, 0, i, 0)),
                  pl.BlockSpec((E, E), lambda b, i: (0, 0)),
                  pl.BlockSpec((1, E), lambda b, i: (0, 0))],
        out_specs=pl.BlockSpec((1, rb, E), lambda b, i: (b, i, 0)),
        out_shape=jax.ShapeDtypeStruct((B, N, E), jnp.float32),
    )(merged.reshape(B, H, N, OW), woT3, bo.reshape(1, E))
    return out


def kernel(x, Wq, bq, Wk, bk, Wv, bv, Wo, bo, probes):
    bqkv = jnp.stack([bq, bk, bv], axis=0)
    pk4, q4 = _qkv(x, Wq.T, Wk.T, Wv.T, bqkv)

    m3np = np.zeros((LW, LW), np.float32)
    su3np = np.zeros((LW, LW), np.float32)
    for lvl, cs in enumerate(BUCKET_SIZES):
        s = slice(lvl * P, (lvl + 1) * P)
        m3np[s, s] = _class_matrix(max(1, N // cs))
        su3np[s, s] = np.triu(np.ones((P, P), np.float32), 1)
    m3 = jnp.asarray(m3np).astype(jnp.bfloat16)
    su3 = jnp.asarray(su3np)
    tril = jnp.asarray(np.tril(np.ones((CSB, CSB), np.float32))
                       ).astype(jnp.bfloat16)
    ranks = _ranks(q4, probes, m3, tril, su3)
    ranks_flat = ranks.reshape(-1)

    sorted_all = _sc_sort(ranks_flat, pk4.reshape(BH * N, PK))
    outs = []
    subblk = {256: 256, 64: 128, 16: 128}
    for lvl, cs in enumerate(BUCKET_SIZES):
        outs.append(_attention(cs, 1024, subblk[cs], sorted_all[lvl]))

    merged = _sc_unsort(ranks_flat, *outs)
    return _oproj(merged, Wo.T * (1.0 / NL), bo)
